# baseline (speedup 1.0000x reference)
import functools

import jax
import jax.numpy as jnp
from jax import lax
from jax.experimental import pallas as pl
from jax.experimental.pallas import tpu as pltpu
from jax.experimental.pallas import tpu_sc as plsc

N_DIM = 32
ROW_W = 128
K = 8


def _make_gather(n_rows: int, n_dim: int):
    info = plsc.get_sparse_core_info()
    nc, ns = info.num_cores, info.num_subcores
    nw = nc * ns
    rows_per_w = n_rows // nw
    n_chunks = rows_per_w // K
    n_pairs = n_chunks // 2
    chunk_elems = K * ROW_W

    mesh = plsc.VectorSubcoreMesh(core_axis_name="c", subcore_axis_name="s")

    @functools.partial(
        pl.kernel,
        mesh=mesh,
        out_type=jax.ShapeDtypeStruct((n_rows * ROW_W, n_dim), jnp.float32),
        scratch_types=[
            pltpu.VMEM((2 * K, ROW_W), jnp.int32),
            pltpu.VMEM((chunk_elems, n_dim), jnp.float32),
            pltpu.VMEM((chunk_elems, n_dim), jnp.float32),
            pltpu.SemaphoreType.DMA,
            pltpu.SemaphoreType.DMA,
            pltpu.SemaphoreType.DMA,
            pltpu.SemaphoreType.DMA,
        ],
        compiler_params=pltpu.CompilerParams(use_tc_tiling_on_sc=False),
    )
    def gather_kernel(idx_hbm, table_hbm, out_hbm, idx_v, rows0, rows1,
                      semg0, semg1, semw0, semw1):
        wid = lax.axis_index("s") * nc + lax.axis_index("c")
        w_row0 = wid * rows_per_w

        def load_idx(pair):
            pltpu.sync_copy(idx_hbm.at[pl.ds(w_row0 + pair * 2 * K, 2 * K)],
                            idx_v)

        def fire_gathers(rows_v, sem, j0):
            return [pltpu.async_copy(
                table_hbm.at[idx_v.at[j0 + j]],
                rows_v.at[pl.ds(j * ROW_W, ROW_W)],
                sem,
            ) for j in range(K)]

        def fire_write(rows_v, sem, chunk):
            return pltpu.async_copy(
                rows_v, out_hbm.at[pl.ds((w_row0 + chunk * K) * ROW_W,
                                         chunk_elems)], sem)

        def wait_write(rows_v, sem):
            pltpu.make_async_copy(
                rows_v, out_hbm.at[pl.ds(0, chunk_elems)], sem).wait()

        load_idx(0)
        g0 = fire_gathers(rows0, semg0, 0)
        g1 = fire_gathers(rows1, semg1, K)
        for c in g0:
            c.wait()
        fire_write(rows0, semw0, 0)
        for c in g1:
            c.wait()
        fire_write(rows1, semw1, 1)

        def pair_body(p, carry):
            wait_write(rows0, semw0)
            load_idx(p)
            g0 = fire_gathers(rows0, semg0, 0)
            wait_write(rows1, semw1)
            g1 = fire_gathers(rows1, semg1, K)
            for c in g0:
                c.wait()
            fire_write(rows0, semw0, 2 * p)
            for c in g1:
                c.wait()
            fire_write(rows1, semw1, 2 * p + 1)
            return carry

        lax.fori_loop(1, n_pairs, pair_body, 0)
        wait_write(rows0, semw0)
        wait_write(rows1, semw1)

    return gather_kernel


def kernel(node_ids, emb_table):
    b, h = node_ids.shape
    n_nodes, n_dim = emb_table.shape
    n_rows = (b * h) // ROW_W
    idx2d = node_ids.reshape(n_rows, ROW_W).astype(jnp.int32)
    out = _make_gather(n_rows, n_dim)(idx2d, emb_table)
    return out.reshape(b, h, n_dim)


# speedup vs baseline: 1.0104x; 1.0104x over previous
import functools

import jax
import jax.numpy as jnp
from jax import lax
from jax.experimental import pallas as pl
from jax.experimental.pallas import tpu as pltpu
from jax.experimental.pallas import tpu_sc as plsc

G = 8
S0 = 128
S1 = 72


def _make_gather(batch: int, hist: int, n_dim: int):
    info = plsc.get_sparse_core_info()
    nc, ns = info.num_cores, info.num_subcores
    nw = nc * ns
    b_per_w = batch // nw
    n_chunks = b_per_w // G
    n_pairs = n_chunks // 2

    mesh = plsc.VectorSubcoreMesh(core_axis_name="c", subcore_axis_name="s")

    @functools.partial(
        pl.kernel,
        mesh=mesh,
        out_type=jax.ShapeDtypeStruct((batch, hist, n_dim), jnp.float32),
        scratch_types=[
            pltpu.VMEM((2 * G, hist), jnp.int32),
            pltpu.VMEM((G, hist, n_dim), jnp.float32),
            pltpu.VMEM((G, hist, n_dim), jnp.float32),
            pltpu.SemaphoreType.DMA,
            pltpu.SemaphoreType.DMA,
            pltpu.SemaphoreType.DMA,
            pltpu.SemaphoreType.DMA,
        ],
        compiler_params=pltpu.CompilerParams(use_tc_tiling_on_sc=False),
    )
    def gather_kernel(idx_hbm, table_hbm, out_hbm, idx_v, rows0, rows1,
                      semg0, semg1, semw0, semw1):
        wid = lax.axis_index("s") * nc + lax.axis_index("c")
        w_b0 = wid * b_per_w

        def load_idx(pair):
            pltpu.sync_copy(idx_hbm.at[pl.ds(w_b0 + pair * 2 * G, 2 * G)],
                            idx_v)

        def fire_gathers(rows_v, sem, g0):
            copies = []
            for g in range(G):
                copies.append(pltpu.async_copy(
                    table_hbm.at[idx_v.at[g0 + g, pl.ds(0, S0)]],
                    rows_v.at[g, pl.ds(0, S0)], sem))
                copies.append(pltpu.async_copy(
                    table_hbm.at[idx_v.at[g0 + g, pl.ds(S0, S1)]],
                    rows_v.at[g, pl.ds(S0, S1)], sem))
            return copies

        def fire_write(rows_v, sem, chunk):
            return pltpu.async_copy(
                rows_v, out_hbm.at[pl.ds(w_b0 + chunk * G, G)], sem)

        def wait_write(rows_v, sem):
            pltpu.make_async_copy(
                rows_v, out_hbm.at[pl.ds(0, G)], sem).wait()

        load_idx(0)
        g0 = fire_gathers(rows0, semg0, 0)
        g1 = fire_gathers(rows1, semg1, G)
        for c in g0:
            c.wait()
        fire_write(rows0, semw0, 0)
        for c in g1:
            c.wait()
        fire_write(rows1, semw1, 1)

        def pair_body(p, carry):
            wait_write(rows0, semw0)
            load_idx(p)
            g0 = fire_gathers(rows0, semg0, 0)
            wait_write(rows1, semw1)
            g1 = fire_gathers(rows1, semg1, G)
            for c in g0:
                c.wait()
            fire_write(rows0, semw0, 2 * p)
            for c in g1:
                c.wait()
            fire_write(rows1, semw1, 2 * p + 1)
            return carry

        lax.fori_loop(1, n_pairs, pair_body, 0)
        wait_write(rows0, semw0)
        wait_write(rows1, semw1)

    return gather_kernel


def kernel(node_ids, emb_table):
    b, h = node_ids.shape
    n_nodes, n_dim = emb_table.shape
    return _make_gather(b, h, n_dim)(node_ids.astype(jnp.int32), emb_table)
